# baseline (speedup 1.0000x reference)
import functools

import jax
import jax.numpy as jnp
from jax import lax
from jax.experimental import pallas as pl
from jax.experimental.pallas import tpu as pltpu
from jax.experimental.pallas import tpu_sc as plsc

_SAMPLE_N = 16384
_FEAT = 128
_NC = 2
_NS = 16
_NW = _NC * _NS
_ROWS_PER_W = _SAMPLE_N // _NW


def _make_sc_kernel():
    mesh = plsc.VectorSubcoreMesh(core_axis_name="c", subcore_axis_name="s")

    n_chunk = 4
    chunk = _ROWS_PER_W // n_chunk

    @functools.partial(
        pl.kernel,
        mesh=mesh,
        out_type=jax.ShapeDtypeStruct((_SAMPLE_N, _FEAT), jnp.float32),
        scratch_types=[
            pltpu.VMEM((2, chunk, _FEAT), jnp.float32),
            pltpu.SemaphoreType.DMA((2,)),
            pltpu.SemaphoreType.DMA((2,)),
        ],
    )
    def k(ds_hbm, out_hbm, buf, in_sems, out_sems):
        wid = lax.axis_index("s") * _NC + lax.axis_index("c")
        base = wid * _ROWS_PER_W

        def in_copy(i, slot):
            return pltpu.make_async_copy(
                ds_hbm.at[pl.ds(base + i * chunk, chunk), :],
                buf.at[slot],
                in_sems.at[slot],
            )

        def out_copy(i, slot):
            return pltpu.make_async_copy(
                buf.at[slot],
                out_hbm.at[pl.ds(base + i * chunk, chunk), :],
                out_sems.at[slot],
            )

        in_copy(0, 0).start()
        for i in range(n_chunk):
            slot = i % 2
            nxt = (i + 1) % 2
            if i + 1 < n_chunk:
                if i >= 1:
                    out_copy(i - 1, nxt).wait()
                in_copy(i + 1, nxt).start()
            in_copy(i, slot).wait()
            out_copy(i, slot).start()
        out_copy(n_chunk - 2, (n_chunk - 2) % 2).wait()
        out_copy(n_chunk - 1, (n_chunk - 1) % 2).wait()

    return k


_sc_kernel = _make_sc_kernel()


def kernel(dataset):
    return _sc_kernel(dataset)


# speedup vs baseline: 3.2023x; 3.2023x over previous
import jax
import jax.numpy as jnp
from jax.experimental import pallas as pl
from jax.experimental.pallas import tpu as pltpu

_SAMPLE_N = 16384
_FEAT = 128
_BLOCK = 4096


def _body(x_ref, o_hbm, sem):
    i = pl.program_id(0)
    cp = pltpu.make_async_copy(
        x_ref, o_hbm.at[pl.ds(i * _BLOCK, _BLOCK), :], sem
    )
    cp.start()
    cp.wait()


def kernel(dataset):
    return pl.pallas_call(
        _body,
        grid=(_SAMPLE_N // _BLOCK,),
        in_specs=[pl.BlockSpec((_BLOCK, _FEAT), lambda i: (i, 0))],
        out_specs=pl.BlockSpec(memory_space=pltpu.MemorySpace.HBM),
        out_shape=jax.ShapeDtypeStruct((_SAMPLE_N, _FEAT), jnp.float32),
        scratch_shapes=[pltpu.SemaphoreType.DMA],
    )(dataset)


# speedup vs baseline: 3.9856x; 1.2446x over previous
import jax
import jax.numpy as jnp
from jax.experimental import pallas as pl
from jax.experimental.pallas import tpu as pltpu

_SAMPLE_N = 16384
_FEAT = 128
_BLOCK = 8192


def _body(x_ref, o_hbm, sem):
    i = pl.program_id(0)
    cp = pltpu.make_async_copy(
        x_ref, o_hbm.at[pl.ds(i * _BLOCK, _BLOCK), :], sem
    )
    cp.start()
    cp.wait()


def kernel(dataset):
    return pl.pallas_call(
        _body,
        grid=(_SAMPLE_N // _BLOCK,),
        in_specs=[pl.BlockSpec((_BLOCK, _FEAT), lambda i: (i, 0))],
        out_specs=pl.BlockSpec(memory_space=pltpu.MemorySpace.HBM),
        out_shape=jax.ShapeDtypeStruct((_SAMPLE_N, _FEAT), jnp.float32),
        scratch_shapes=[pltpu.SemaphoreType.DMA],
    )(dataset)


# speedup vs baseline: 4.3660x; 1.0955x over previous
import jax
import jax.numpy as jnp
from jax.experimental import pallas as pl

_SAMPLE_N = 16384
_FEAT = 128
_BLOCK = 8192


def _copy_body(x_ref, o_ref):
    o_ref[...] = x_ref[...]


def kernel(dataset):
    return pl.pallas_call(
        _copy_body,
        grid=(_SAMPLE_N // _BLOCK,),
        in_specs=[pl.BlockSpec((_BLOCK, _FEAT), lambda i: (i, 0))],
        out_specs=pl.BlockSpec((_BLOCK, _FEAT), lambda i: (i, 0)),
        out_shape=jax.ShapeDtypeStruct((_SAMPLE_N, _FEAT), jnp.float32),
    )(dataset)
